# baseline (speedup 1.0000x reference)
import functools

import jax
import jax.numpy as jnp
from jax import lax
from jax.experimental import pallas as pl
from jax.experimental.pallas import tpu as pltpu
from jax.experimental.pallas import tpu_sc as plsc

N = 10000
E = 320000
NC, NS, LANES = 2, 16, 16
NW = NC * NS
K = 128
NCHUNK = 80
EP = NW * NCHUNK * K
G = 4
GD = 8
STRIPE = 640
NPAD = NS * STRIPE
ZB = 16

_mesh = plsc.VectorSubcoreMesh(core_axis_name="c", subcore_axis_name="s")


@functools.partial(
    pl.kernel,
    mesh=_mesh,
    out_type=jax.ShapeDtypeStruct((NC, NPAD), jnp.float32),
    scratch_types=[
        pltpu.VMEM((NCHUNK, K), jnp.int32),
        pltpu.VMEM((K,), jnp.float32),
        pltpu.VMEM((STRIPE,), jnp.float32),
        pltpu.VMEM_SHARED((NPAD,), jnp.float32),
        pltpu.SemaphoreType.DMA,
    ],
    compiler_params=pltpu.CompilerParams(use_tc_tiling_on_sc=False),
)
def _sc_degree(dst2_hbm, out_hbm, didx, ones_v, zero_v, acc_sh, sem):
    c = lax.axis_index("c")
    s = lax.axis_index("s")
    wid = s * NC + c
    for i in range(K // LANES):
        ones_v[pl.ds(i * LANES, LANES)] = jnp.ones((LANES,), jnp.float32)
    for i in range(STRIPE // LANES):
        zero_v[pl.ds(i * LANES, LANES)] = jnp.zeros((LANES,), jnp.float32)
    pltpu.sync_copy(dst2_hbm.at[pl.ds(wid * NCHUNK, NCHUNK)], didx)
    pltpu.sync_copy(zero_v, acc_sh.at[pl.ds(s * STRIPE, STRIPE)])
    plsc.subcore_barrier()

    def body(t, carry):
        cps = []
        for b in range(GD):
            j = t * GD + b
            cps.append(pltpu.async_copy(
                ones_v, acc_sh.at[didx.at[j]], sem, add=True))
        for cp in cps:
            cp.wait()
        return carry

    lax.fori_loop(0, NCHUNK // GD, body, 0)
    plsc.subcore_barrier()
    pltpu.sync_copy(acc_sh.at[pl.ds(s * STRIPE, STRIPE)],
                    out_hbm.at[c, pl.ds(s * STRIPE, STRIPE)])


def _make_sc_aggregate(D):
    SL = NCHUNK
    ZR = 160

    @functools.partial(
        pl.kernel,
        mesh=_mesh,
        out_type=jax.ShapeDtypeStruct((NC, NPAD, D), jnp.float32),
        scratch_types=[
            pltpu.VMEM((SL, K), jnp.int32),
            pltpu.VMEM((SL, K), jnp.int32),
            [pltpu.VMEM((K, D), jnp.float32) for _ in range(G)],
            pltpu.VMEM((ZR, D), jnp.float32),
            pltpu.VMEM_SHARED((NPAD, D), jnp.float32),
            [pltpu.SemaphoreType.DMA for _ in range(G)],
            [pltpu.SemaphoreType.DMA for _ in range(2)],
        ],
        compiler_params=pltpu.CompilerParams(use_tc_tiling_on_sc=False),
    )
    def agg(h_hbm, src2_hbm, dst2_hbm, out_hbm, sidx, didx, rows, ztile, acc,
            gsem, ssem):
        c = lax.axis_index("c")
        s = lax.axis_index("s")
        wid = s * NC + c
        pltpu.sync_copy(src2_hbm.at[pl.ds(wid * SL, SL)], sidx)
        pltpu.sync_copy(dst2_hbm.at[pl.ds(wid * SL, SL)], didx)
        for b in range(G):
            pltpu.async_copy(h_hbm.at[sidx.at[b]], rows[b], gsem[b])
        for i in range(ZR):
            for j in range(D // LANES):
                ztile[i, pl.ds(j * LANES, LANES)] = jnp.zeros((LANES,), jnp.float32)
        for t in range(STRIPE // ZR):
            pltpu.sync_copy(ztile, acc.at[pl.ds(s * STRIPE + t * ZR, ZR)])
        plsc.subcore_barrier()

        NT = SL // G

        def body(t, carry):
            for b in range(G):
                j = t * G + b
                pltpu.make_async_copy(h_hbm.at[sidx.at[j]], rows[b], gsem[b]).wait()
                pltpu.async_copy(rows[b], acc.at[didx.at[j]], ssem[b % 2], add=True)
                bp = (b - 1) % G

                @pl.when(jnp.logical_and(j >= 1, j <= SL - G))
                def _():
                    pltpu.make_async_copy(
                        rows[bp], acc.at[didx.at[j - 1]], ssem[(b - 1) % 2]).wait()
                    pltpu.async_copy(h_hbm.at[sidx.at[j + G - 1]], rows[bp], gsem[bp])

                @pl.when(j > SL - G)
                def _():
                    pltpu.make_async_copy(
                        rows[bp], acc.at[didx.at[j - 1]], ssem[(b - 1) % 2]).wait()
            return carry

        lax.fori_loop(0, NT, body, 0)
        pltpu.make_async_copy(
            rows[(SL - 1) % G], acc.at[didx.at[SL - 1]],
            ssem[(SL - 1) % 2]).wait()
        plsc.subcore_barrier()
        pltpu.sync_copy(acc.at[pl.ds(s * STRIPE, STRIPE)],
                        out_hbm.at[c, pl.ds(s * STRIPE, STRIPE)])

    return agg


_sc_agg64 = _make_sc_aggregate(64)
_sc_agg16 = _make_sc_aggregate(16)

RB = 2000


def _dinv_of(deg_ref):
    return lax.rsqrt(deg_ref[:, 0:1] + deg_ref[:, 1:2] + 1.0)


def _tc_layer1(degT, x, W1):
    def body(deg_ref, x_ref, w_ref, o_ref):
        dinv = _dinv_of(deg_ref)
        o_ref[...] = jnp.dot(x_ref[...], w_ref[...],
                             preferred_element_type=jnp.float32) * dinv

    return pl.pallas_call(
        body,
        grid=(N // RB,),
        in_specs=[
            pl.BlockSpec((RB, 2), lambda i: (i, 0)),
            pl.BlockSpec((RB, 128), lambda i: (i, 0)),
            pl.BlockSpec((128, 64), lambda i: (0, 0)),
        ],
        out_specs=pl.BlockSpec((RB, 64), lambda i: (i, 0)),
        out_shape=jax.ShapeDtypeStruct((N, 64), jnp.float32),
    )(degT, x, W1)


def _tc_mid(degT, p, h1p, b1, W2):
    def body(deg_ref, p_ref, h_ref, b_ref, w_ref, o_ref):
        dinv = _dinv_of(deg_ref)
        z = (p_ref[0] + p_ref[1] + h_ref[...]) * dinv + b_ref[...]
        z = jnp.maximum(z, 0.0)
        o_ref[...] = jnp.dot(z, w_ref[...],
                             preferred_element_type=jnp.float32) * dinv

    return pl.pallas_call(
        body,
        grid=(N // RB,),
        in_specs=[
            pl.BlockSpec((RB, 2), lambda i: (i, 0)),
            pl.BlockSpec((2, RB, 64), lambda i: (0, i, 0)),
            pl.BlockSpec((RB, 64), lambda i: (i, 0)),
            pl.BlockSpec((1, 64), lambda i: (0, 0)),
            pl.BlockSpec((64, 16), lambda i: (0, 0)),
        ],
        out_specs=pl.BlockSpec((RB, 16), lambda i: (i, 0)),
        out_shape=jax.ShapeDtypeStruct((N, 16), jnp.float32),
    )(degT, p, h1p, b1, W2)


def _tc_final(degT, q, h2p, b2):
    def body(deg_ref, q_ref, h_ref, b_ref, o_ref):
        dinv = _dinv_of(deg_ref)
        o_ref[...] = (q_ref[0] + q_ref[1] + h_ref[...]) * dinv + b_ref[...]

    return pl.pallas_call(
        body,
        grid=(N // RB,),
        in_specs=[
            pl.BlockSpec((RB, 2), lambda i: (i, 0)),
            pl.BlockSpec((2, RB, 16), lambda i: (0, i, 0)),
            pl.BlockSpec((RB, 16), lambda i: (i, 0)),
            pl.BlockSpec((1, 16), lambda i: (0, 0)),
        ],
        out_specs=pl.BlockSpec((RB, 16), lambda i: (i, 0)),
        out_shape=jax.ShapeDtypeStruct((N, 16), jnp.float32),
    )(degT, q, h2p, b2)


def kernel(x, edge_index, W1, b1, W2, b2):
    src2 = jnp.pad(edge_index[0], (0, EP - E)).reshape(EP // K, K)
    dst2 = jnp.pad(edge_index[1], (0, EP - E),
                   constant_values=NPAD - 1).reshape(EP // K, K)
    degp = _sc_degree(dst2)
    degT = jnp.transpose(degp)[:N]
    h1p = _tc_layer1(degT, x, W1)
    p = _sc_agg64(h1p, src2, dst2)
    h2p = _tc_mid(degT, p, h1p, b1.reshape(1, 64), W2)
    q = _sc_agg16(h2p, src2, dst2)
    out = _tc_final(degT, q, h2p, b2.reshape(1, 16))
    return (out, edge_index)


# speedup vs baseline: 2.0812x; 2.0812x over previous
import functools

import jax
import jax.numpy as jnp
from jax import lax
from jax.experimental import pallas as pl
from jax.experimental.pallas import tpu as pltpu
from jax.experimental.pallas import tpu_sc as plsc

N = 10000
E = 320000
NC, NS, LANES = 2, 16, 16
NW = NC * NS
K = 128
NCHUNK = 80
EP = NW * NCHUNK * K
G = 4
GD = 8
STRIPE = 640
NPAD = NS * STRIPE
ZB = 16

_mesh = plsc.VectorSubcoreMesh(core_axis_name="c", subcore_axis_name="s")


@functools.partial(
    pl.kernel,
    mesh=_mesh,
    out_type=jax.ShapeDtypeStruct((NC, NPAD), jnp.float32),
    scratch_types=[
        pltpu.VMEM((NCHUNK, K), jnp.int32),
        pltpu.VMEM((K,), jnp.float32),
        pltpu.VMEM((STRIPE,), jnp.float32),
        pltpu.VMEM_SHARED((NPAD,), jnp.float32),
        pltpu.SemaphoreType.DMA,
    ],
    compiler_params=pltpu.CompilerParams(use_tc_tiling_on_sc=False),
)
def _sc_degree(dst2_hbm, out_hbm, didx, ones_v, zero_v, acc_sh, sem):
    c = lax.axis_index("c")
    s = lax.axis_index("s")
    wid = s * NC + c
    for i in range(K // LANES):
        ones_v[pl.ds(i * LANES, LANES)] = jnp.ones((LANES,), jnp.float32)
    for i in range(STRIPE // LANES):
        zero_v[pl.ds(i * LANES, LANES)] = jnp.zeros((LANES,), jnp.float32)
    pltpu.sync_copy(dst2_hbm.at[pl.ds(wid * NCHUNK, NCHUNK)], didx)
    pltpu.sync_copy(zero_v, acc_sh.at[pl.ds(s * STRIPE, STRIPE)])
    plsc.subcore_barrier()

    def body(t, carry):
        cps = []
        for b in range(GD):
            j = t * GD + b
            cps.append(pltpu.async_copy(
                ones_v, acc_sh.at[didx.at[j]], sem, add=True))
        for cp in cps:
            cp.wait()
        return carry

    lax.fori_loop(0, NCHUNK // GD, body, 0)
    plsc.subcore_barrier()
    pltpu.sync_copy(acc_sh.at[pl.ds(s * STRIPE, STRIPE)],
                    out_hbm.at[c, pl.ds(s * STRIPE, STRIPE)])


def _make_sc_aggregate(D):
    SL = NCHUNK
    ZR = 160

    @functools.partial(
        pl.kernel,
        mesh=_mesh,
        out_type=jax.ShapeDtypeStruct((NC, NPAD, D), jnp.float32),
        scratch_types=[
            pltpu.VMEM((SL, K), jnp.int32),
            pltpu.VMEM((SL, K), jnp.int32),
            [pltpu.VMEM((K, D), jnp.float32) for _ in range(G)],
            pltpu.VMEM((ZR, D), jnp.float32),
            pltpu.VMEM_SHARED((NPAD, D), jnp.float32),
            [pltpu.SemaphoreType.DMA for _ in range(G)],
            [pltpu.SemaphoreType.DMA for _ in range(2)],
        ],
        compiler_params=pltpu.CompilerParams(use_tc_tiling_on_sc=False),
    )
    def agg(h_hbm, src2_hbm, dst2_hbm, out_hbm, sidx, didx, rows, ztile, acc,
            gsem, ssem):
        c = lax.axis_index("c")
        s = lax.axis_index("s")
        wid = s * NC + c
        pltpu.sync_copy(src2_hbm.at[pl.ds(wid * SL, SL)], sidx)
        pltpu.sync_copy(dst2_hbm.at[pl.ds(wid * SL, SL)], didx)
        for b in range(G):
            pltpu.async_copy(h_hbm.at[sidx.at[b]], rows[b], gsem[b])
        for i in range(ZR):
            for j in range(D // LANES):
                ztile[i, pl.ds(j * LANES, LANES)] = jnp.zeros((LANES,), jnp.float32)
        for t in range(STRIPE // ZR):
            pltpu.sync_copy(ztile, acc.at[pl.ds(s * STRIPE + t * ZR, ZR)])
        plsc.subcore_barrier()

        NT = SL // G

        def body(t, carry):
            for b in range(G):
                j = t * G + b
                pltpu.make_async_copy(h_hbm.at[sidx.at[j]], rows[b], gsem[b]).wait()
                pltpu.async_copy(rows[b], acc.at[didx.at[j]], ssem[b % 2], add=True)
                bp = (b - 1) % G

                @pl.when(jnp.logical_and(j >= 1, j <= SL - G))
                def _():
                    pltpu.make_async_copy(
                        rows[bp], acc.at[didx.at[j - 1]], ssem[(b - 1) % 2]).wait()
                    pltpu.async_copy(h_hbm.at[sidx.at[j + G - 1]], rows[bp], gsem[bp])

                @pl.when(j > SL - G)
                def _():
                    pltpu.make_async_copy(
                        rows[bp], acc.at[didx.at[j - 1]], ssem[(b - 1) % 2]).wait()
            return carry

        lax.fori_loop(0, NT, body, 0)
        pltpu.make_async_copy(
            rows[(SL - 1) % G], acc.at[didx.at[SL - 1]],
            ssem[(SL - 1) % 2]).wait()
        plsc.subcore_barrier()
        pltpu.sync_copy(acc.at[pl.ds(s * STRIPE, STRIPE)],
                        out_hbm.at[c, pl.ds(s * STRIPE, STRIPE)])

    return agg


_sc_agg64 = _make_sc_aggregate(64)
_sc_agg16 = _make_sc_aggregate(16)

RB = 2000


def _dinv_of(deg_ref):
    return lax.rsqrt(deg_ref[:, 0:1] + deg_ref[:, 1:2] + 1.0)


def _tc_layer1(degT, x, W1):
    def body(deg_ref, x_ref, w_ref, o_ref):
        dinv = _dinv_of(deg_ref)
        o_ref[...] = jnp.dot(x_ref[...], w_ref[...],
                             preferred_element_type=jnp.float32) * dinv

    return pl.pallas_call(
        body,
        grid=(N // RB,),
        in_specs=[
            pl.BlockSpec((RB, 2), lambda i: (i, 0)),
            pl.BlockSpec((RB, 128), lambda i: (i, 0)),
            pl.BlockSpec((128, 64), lambda i: (0, 0)),
        ],
        out_specs=pl.BlockSpec((RB, 64), lambda i: (i, 0)),
        out_shape=jax.ShapeDtypeStruct((N, 64), jnp.float32),
    )(degT, x, W1)


def _tc_mid(degT, p, h1p, b1, W2):
    def body(deg_ref, p_ref, h_ref, b_ref, w_ref, o_ref):
        dinv = _dinv_of(deg_ref)
        z = (p_ref[0] + p_ref[1] + h_ref[...]) * dinv + b_ref[...]
        z = jnp.maximum(z, 0.0)
        o_ref[...] = jnp.dot(z, w_ref[...],
                             preferred_element_type=jnp.float32) * dinv

    return pl.pallas_call(
        body,
        grid=(N // RB,),
        in_specs=[
            pl.BlockSpec((RB, 2), lambda i: (i, 0)),
            pl.BlockSpec((2, RB, 64), lambda i: (0, i, 0)),
            pl.BlockSpec((RB, 64), lambda i: (i, 0)),
            pl.BlockSpec((1, 64), lambda i: (0, 0)),
            pl.BlockSpec((64, 16), lambda i: (0, 0)),
        ],
        out_specs=pl.BlockSpec((RB, 16), lambda i: (i, 0)),
        out_shape=jax.ShapeDtypeStruct((N, 16), jnp.float32),
    )(degT, p, h1p, b1, W2)


def _tc_final(degT, q, h2p, b2):
    def body(deg_ref, q_ref, h_ref, b_ref, o_ref):
        dinv = _dinv_of(deg_ref)
        o_ref[...] = (q_ref[0] + q_ref[1] + h_ref[...]) * dinv + b_ref[...]

    return pl.pallas_call(
        body,
        grid=(N // RB,),
        in_specs=[
            pl.BlockSpec((RB, 2), lambda i: (i, 0)),
            pl.BlockSpec((2, RB, 16), lambda i: (0, i, 0)),
            pl.BlockSpec((RB, 16), lambda i: (i, 0)),
            pl.BlockSpec((1, 16), lambda i: (0, 0)),
        ],
        out_specs=pl.BlockSpec((RB, 16), lambda i: (i, 0)),
        out_shape=jax.ShapeDtypeStruct((N, 16), jnp.float32),
    )(degT, q, h2p, b2)


def kernel(x, edge_index, W1, b1, W2, b2):
    pidx = jnp.arange(EP - E, dtype=jnp.int32)
    src2 = jnp.concatenate([edge_index[0], pidx % N]).reshape(EP // K, K)
    dst2 = jnp.concatenate([edge_index[1],
                            N + pidx % (NPAD - N)]).reshape(EP // K, K)
    degp = _sc_degree(dst2)
    degT = jnp.transpose(degp)[:N]
    h1p = _tc_layer1(degT, x, W1)
    p = _sc_agg64(h1p, src2, dst2)
    h2p = _tc_mid(degT, p, h1p, b1.reshape(1, 64), W2)
    q = _sc_agg16(h2p, src2, dst2)
    out = _tc_final(degT, q, h2p, b2.reshape(1, 16))
    return (out, edge_index)


# speedup vs baseline: 2.1874x; 1.0510x over previous
import functools

import jax
import jax.numpy as jnp
from jax import lax
from jax.experimental import pallas as pl
from jax.experimental.pallas import tpu as pltpu
from jax.experimental.pallas import tpu_sc as plsc

N = 10000
E = 320000
NC, NS, LANES = 2, 16, 16
NW = NC * NS
K = 128
NCHUNK = 80
EP = NW * NCHUNK * K

GD = 8
STRIPE = 640
NPAD = NS * STRIPE
ZB = 16

_mesh = plsc.VectorSubcoreMesh(core_axis_name="c", subcore_axis_name="s")


@functools.partial(
    pl.kernel,
    mesh=_mesh,
    out_type=jax.ShapeDtypeStruct((NC, NPAD), jnp.float32),
    scratch_types=[
        pltpu.VMEM((NCHUNK, K), jnp.int32),
        pltpu.VMEM((K,), jnp.float32),
        pltpu.VMEM((STRIPE,), jnp.float32),
        pltpu.VMEM_SHARED((NPAD,), jnp.float32),
        pltpu.SemaphoreType.DMA,
    ],
    compiler_params=pltpu.CompilerParams(use_tc_tiling_on_sc=False),
)
def _sc_degree(dst2_hbm, out_hbm, didx, ones_v, zero_v, acc_sh, sem):
    c = lax.axis_index("c")
    s = lax.axis_index("s")
    wid = s * NC + c
    for i in range(K // LANES):
        ones_v[pl.ds(i * LANES, LANES)] = jnp.ones((LANES,), jnp.float32)
    for i in range(STRIPE // LANES):
        zero_v[pl.ds(i * LANES, LANES)] = jnp.zeros((LANES,), jnp.float32)
    pltpu.sync_copy(dst2_hbm.at[pl.ds(wid * NCHUNK, NCHUNK)], didx)
    pltpu.sync_copy(zero_v, acc_sh.at[pl.ds(s * STRIPE, STRIPE)])
    plsc.subcore_barrier()

    def body(t, carry):
        cps = []
        for b in range(GD):
            j = t * GD + b
            cps.append(pltpu.async_copy(
                ones_v, acc_sh.at[didx.at[j]], sem, add=True))
        for cp in cps:
            cp.wait()
        return carry

    lax.fori_loop(0, NCHUNK // GD, body, 0)
    plsc.subcore_barrier()
    pltpu.sync_copy(acc_sh.at[pl.ds(s * STRIPE, STRIPE)],
                    out_hbm.at[c, pl.ds(s * STRIPE, STRIPE)])


def _make_sc_aggregate(D, G):
    SL = NCHUNK
    ZR = 160

    @functools.partial(
        pl.kernel,
        mesh=_mesh,
        out_type=jax.ShapeDtypeStruct((NC, NPAD, D), jnp.float32),
        scratch_types=[
            pltpu.VMEM((SL, K), jnp.int32),
            pltpu.VMEM((SL, K), jnp.int32),
            [pltpu.VMEM((K, D), jnp.float32) for _ in range(G)],
            pltpu.VMEM((ZR, D), jnp.float32),
            pltpu.VMEM_SHARED((NPAD, D), jnp.float32),
            [pltpu.SemaphoreType.DMA for _ in range(G)],
            [pltpu.SemaphoreType.DMA for _ in range(2)],
        ],
        compiler_params=pltpu.CompilerParams(use_tc_tiling_on_sc=False),
    )
    def agg(h_hbm, src2_hbm, dst2_hbm, out_hbm, sidx, didx, rows, ztile, acc,
            gsem, ssem):
        c = lax.axis_index("c")
        s = lax.axis_index("s")
        wid = s * NC + c
        pltpu.sync_copy(src2_hbm.at[pl.ds(wid * SL, SL)], sidx)
        pltpu.sync_copy(dst2_hbm.at[pl.ds(wid * SL, SL)], didx)
        for b in range(G):
            pltpu.async_copy(h_hbm.at[sidx.at[b]], rows[b], gsem[b])
        for i in range(ZR):
            for j in range(D // LANES):
                ztile[i, pl.ds(j * LANES, LANES)] = jnp.zeros((LANES,), jnp.float32)
        for t in range(STRIPE // ZR):
            pltpu.sync_copy(ztile, acc.at[pl.ds(s * STRIPE + t * ZR, ZR)])
        plsc.subcore_barrier()

        NT = SL // G

        def body(t, carry):
            for b in range(G):
                j = t * G + b
                pltpu.make_async_copy(
                    h_hbm.at[sidx.at[j]], rows[b], gsem[b]).wait()
                pltpu.async_copy(rows[b], acc.at[didx.at[j]],
                                 ssem[b % 2], add=True)
                bp = (b - 1) % G

                @pl.when(jnp.logical_and(j >= 1, j <= SL - G))
                def _():
                    pltpu.make_async_copy(
                        rows[bp], acc.at[didx.at[j - 1]],
                        ssem[(b - 1) % 2]).wait()
                    pltpu.async_copy(
                        h_hbm.at[sidx.at[j + G - 1]], rows[bp],
                        gsem[bp])

                @pl.when(j > SL - G)
                def _():
                    pltpu.make_async_copy(
                        rows[bp], acc.at[didx.at[j - 1]],
                        ssem[(b - 1) % 2]).wait()
            return carry

        lax.fori_loop(0, NT, body, 0)
        pltpu.make_async_copy(
            rows[(SL - 1) % G], acc.at[didx.at[SL - 1]],
            ssem[(SL - 1) % 2]).wait()
        plsc.subcore_barrier()
        pltpu.sync_copy(acc.at[pl.ds(s * STRIPE, STRIPE)],
                        out_hbm.at[c, pl.ds(s * STRIPE, STRIPE)])

    return agg


_sc_agg64 = _make_sc_aggregate(64, 4)
_sc_agg16 = _make_sc_aggregate(16, 8)

RB = 2000


def _dinv_of(deg_ref):
    return lax.rsqrt(deg_ref[:, 0:1] + deg_ref[:, 1:2] + 1.0)


def _tc_layer1(degT, x, W1):
    def body(deg_ref, x_ref, w_ref, o_ref):
        dinv = _dinv_of(deg_ref)
        o_ref[...] = jnp.dot(x_ref[...], w_ref[...],
                             preferred_element_type=jnp.float32) * dinv

    return pl.pallas_call(
        body,
        grid=(N // RB,),
        in_specs=[
            pl.BlockSpec((RB, 2), lambda i: (i, 0)),
            pl.BlockSpec((RB, 128), lambda i: (i, 0)),
            pl.BlockSpec((128, 64), lambda i: (0, 0)),
        ],
        out_specs=pl.BlockSpec((RB, 64), lambda i: (i, 0)),
        out_shape=jax.ShapeDtypeStruct((N, 64), jnp.float32),
    )(degT, x, W1)


def _tc_mid(degT, p, h1p, b1, W2):
    def body(deg_ref, p_ref, h_ref, b_ref, w_ref, o_ref):
        dinv = _dinv_of(deg_ref)
        z = (p_ref[0] + p_ref[1] + h_ref[...]) * dinv + b_ref[...]
        z = jnp.maximum(z, 0.0)
        o_ref[...] = jnp.dot(z, w_ref[...],
                             preferred_element_type=jnp.float32) * dinv

    return pl.pallas_call(
        body,
        grid=(N // RB,),
        in_specs=[
            pl.BlockSpec((RB, 2), lambda i: (i, 0)),
            pl.BlockSpec((2, RB, 64), lambda i: (0, i, 0)),
            pl.BlockSpec((RB, 64), lambda i: (i, 0)),
            pl.BlockSpec((1, 64), lambda i: (0, 0)),
            pl.BlockSpec((64, 16), lambda i: (0, 0)),
        ],
        out_specs=pl.BlockSpec((RB, 16), lambda i: (i, 0)),
        out_shape=jax.ShapeDtypeStruct((N, 16), jnp.float32),
    )(degT, p, h1p, b1, W2)


def _tc_final(degT, q, h2p, b2):
    def body(deg_ref, q_ref, h_ref, b_ref, o_ref):
        dinv = _dinv_of(deg_ref)
        o_ref[...] = (q_ref[0] + q_ref[1] + h_ref[...]) * dinv + b_ref[...]

    return pl.pallas_call(
        body,
        grid=(N // RB,),
        in_specs=[
            pl.BlockSpec((RB, 2), lambda i: (i, 0)),
            pl.BlockSpec((2, RB, 16), lambda i: (0, i, 0)),
            pl.BlockSpec((RB, 16), lambda i: (i, 0)),
            pl.BlockSpec((1, 16), lambda i: (0, 0)),
        ],
        out_specs=pl.BlockSpec((RB, 16), lambda i: (i, 0)),
        out_shape=jax.ShapeDtypeStruct((N, 16), jnp.float32),
    )(degT, q, h2p, b2)


def kernel(x, edge_index, W1, b1, W2, b2):
    pidx = jnp.arange(EP - E, dtype=jnp.int32)
    src2 = jnp.concatenate([edge_index[0], pidx % N]).reshape(EP // K, K)
    dst2 = jnp.concatenate([edge_index[1],
                            N + pidx % (NPAD - N)]).reshape(EP // K, K)
    degp = _sc_degree(dst2)
    degT = jnp.transpose(degp)[:N]
    h1p = _tc_layer1(degT, x, W1)
    p = _sc_agg64(h1p, src2, dst2)
    h2p = _tc_mid(degT, p, h1p, b1.reshape(1, 64), W2)
    q = _sc_agg16(h2p, src2, dst2)
    out = _tc_final(degT, q, h2p, b2.reshape(1, 16))
    return (out, edge_index)


# speedup vs baseline: 2.2701x; 1.0378x over previous
import functools

import jax
import jax.numpy as jnp
from jax import lax
from jax.experimental import pallas as pl
from jax.experimental.pallas import tpu as pltpu
from jax.experimental.pallas import tpu_sc as plsc

N = 10000
E = 320000
NC, NS, LANES = 2, 16, 16
NW = NC * NS
K = 128
NCHUNK = 80
EP = NW * NCHUNK * K

GD = 16
STRIPE = 640
NPAD = NS * STRIPE
ZB = 16

_mesh = plsc.VectorSubcoreMesh(core_axis_name="c", subcore_axis_name="s")


@functools.partial(
    pl.kernel,
    mesh=_mesh,
    out_type=jax.ShapeDtypeStruct((NC, NPAD), jnp.float32),
    scratch_types=[
        pltpu.VMEM((NCHUNK, K), jnp.int32),
        pltpu.VMEM((K,), jnp.float32),
        pltpu.VMEM((STRIPE,), jnp.float32),
        pltpu.VMEM_SHARED((NPAD,), jnp.float32),
        pltpu.SemaphoreType.DMA,
    ],
    compiler_params=pltpu.CompilerParams(use_tc_tiling_on_sc=False),
)
def _sc_degree(dst2_hbm, out_hbm, didx, ones_v, zero_v, acc_sh, sem):
    c = lax.axis_index("c")
    s = lax.axis_index("s")
    wid = s * NC + c
    for i in range(K // LANES):
        ones_v[pl.ds(i * LANES, LANES)] = jnp.ones((LANES,), jnp.float32)
    for i in range(STRIPE // LANES):
        zero_v[pl.ds(i * LANES, LANES)] = jnp.zeros((LANES,), jnp.float32)
    pltpu.sync_copy(dst2_hbm.at[pl.ds(wid * NCHUNK, NCHUNK)], didx)
    pltpu.sync_copy(zero_v, acc_sh.at[pl.ds(s * STRIPE, STRIPE)])
    plsc.subcore_barrier()

    def body(t, carry):
        cps = []
        for b in range(GD):
            j = t * GD + b
            cps.append(pltpu.async_copy(
                ones_v, acc_sh.at[didx.at[j]], sem, add=True))
        for cp in cps:
            cp.wait()
        return carry

    lax.fori_loop(0, NCHUNK // GD, body, 0)
    plsc.subcore_barrier()
    pltpu.sync_copy(acc_sh.at[pl.ds(s * STRIPE, STRIPE)],
                    out_hbm.at[c, pl.ds(s * STRIPE, STRIPE)])


def _make_sc_aggregate(D, G, ZR):
    SL = NCHUNK

    @functools.partial(
        pl.kernel,
        mesh=_mesh,
        out_type=jax.ShapeDtypeStruct((NC, NPAD, D), jnp.float32),
        scratch_types=[
            pltpu.VMEM((SL, K), jnp.int32),
            pltpu.VMEM((SL, K), jnp.int32),
            [pltpu.VMEM((K, D), jnp.float32) for _ in range(G)],
            pltpu.VMEM((ZR, D), jnp.float32),
            pltpu.VMEM_SHARED((NPAD, D), jnp.float32),
            [pltpu.SemaphoreType.DMA for _ in range(G)],
            [pltpu.SemaphoreType.DMA for _ in range(2)],
        ],
        compiler_params=pltpu.CompilerParams(use_tc_tiling_on_sc=False),
    )
    def agg(h_hbm, src2_hbm, dst2_hbm, out_hbm, sidx, didx, rows, ztile, acc,
            gsem, ssem):
        c = lax.axis_index("c")
        s = lax.axis_index("s")
        wid = s * NC + c
        pltpu.sync_copy(src2_hbm.at[pl.ds(wid * SL, SL)], sidx)
        pltpu.sync_copy(dst2_hbm.at[pl.ds(wid * SL, SL)], didx)
        for b in range(G):
            pltpu.async_copy(h_hbm.at[sidx.at[b]], rows[b], gsem[b])
        for i in range(ZR):
            for j in range(D // LANES):
                ztile[i, pl.ds(j * LANES, LANES)] = jnp.zeros((LANES,), jnp.float32)
        for t in range(STRIPE // ZR):
            pltpu.sync_copy(ztile, acc.at[pl.ds(s * STRIPE + t * ZR, ZR)])
        plsc.subcore_barrier()

        NT = SL // G

        def body(t, carry):
            for b in range(G):
                j = t * G + b
                pltpu.make_async_copy(
                    h_hbm.at[sidx.at[j]], rows[b], gsem[b]).wait()
                pltpu.async_copy(rows[b], acc.at[didx.at[j]],
                                 ssem[b % 2], add=True)
                bp = (b - 1) % G

                @pl.when(jnp.logical_and(j >= 1, j <= SL - G))
                def _():
                    pltpu.make_async_copy(
                        rows[bp], acc.at[didx.at[j - 1]],
                        ssem[(b - 1) % 2]).wait()
                    pltpu.async_copy(
                        h_hbm.at[sidx.at[j + G - 1]], rows[bp],
                        gsem[bp])

                @pl.when(j > SL - G)
                def _():
                    pltpu.make_async_copy(
                        rows[bp], acc.at[didx.at[j - 1]],
                        ssem[(b - 1) % 2]).wait()
            return carry

        lax.fori_loop(0, NT, body, 0)
        pltpu.make_async_copy(
            rows[(SL - 1) % G], acc.at[didx.at[SL - 1]],
            ssem[(SL - 1) % 2]).wait()
        plsc.subcore_barrier()
        pltpu.sync_copy(acc.at[pl.ds(s * STRIPE, STRIPE)],
                        out_hbm.at[c, pl.ds(s * STRIPE, STRIPE)])

    return agg


_sc_agg64 = _make_sc_aggregate(64, 8, 40)
_sc_agg16 = _make_sc_aggregate(16, 8, 160)

RB = 2000


def _dinv_of(deg_ref):
    return lax.rsqrt(deg_ref[:, 0:1] + deg_ref[:, 1:2] + 1.0)


def _tc_layer1(degT, x, W1):
    def body(deg_ref, x_ref, w_ref, o_ref):
        dinv = _dinv_of(deg_ref)
        o_ref[...] = jnp.dot(x_ref[...], w_ref[...],
                             preferred_element_type=jnp.float32) * dinv

    return pl.pallas_call(
        body,
        grid=(N // RB,),
        in_specs=[
            pl.BlockSpec((RB, 2), lambda i: (i, 0)),
            pl.BlockSpec((RB, 128), lambda i: (i, 0)),
            pl.BlockSpec((128, 64), lambda i: (0, 0)),
        ],
        out_specs=pl.BlockSpec((RB, 64), lambda i: (i, 0)),
        out_shape=jax.ShapeDtypeStruct((N, 64), jnp.float32),
    )(degT, x, W1)


def _tc_mid(degT, p, h1p, b1, W2):
    def body(deg_ref, p_ref, h_ref, b_ref, w_ref, o_ref):
        dinv = _dinv_of(deg_ref)
        z = (p_ref[0] + p_ref[1] + h_ref[...]) * dinv + b_ref[...]
        z = jnp.maximum(z, 0.0)
        o_ref[...] = jnp.dot(z, w_ref[...],
                             preferred_element_type=jnp.float32) * dinv

    return pl.pallas_call(
        body,
        grid=(N // RB,),
        in_specs=[
            pl.BlockSpec((RB, 2), lambda i: (i, 0)),
            pl.BlockSpec((2, RB, 64), lambda i: (0, i, 0)),
            pl.BlockSpec((RB, 64), lambda i: (i, 0)),
            pl.BlockSpec((1, 64), lambda i: (0, 0)),
            pl.BlockSpec((64, 16), lambda i: (0, 0)),
        ],
        out_specs=pl.BlockSpec((RB, 16), lambda i: (i, 0)),
        out_shape=jax.ShapeDtypeStruct((N, 16), jnp.float32),
    )(degT, p, h1p, b1, W2)


def _tc_final(degT, q, h2p, b2):
    def body(deg_ref, q_ref, h_ref, b_ref, o_ref):
        dinv = _dinv_of(deg_ref)
        o_ref[...] = (q_ref[0] + q_ref[1] + h_ref[...]) * dinv + b_ref[...]

    return pl.pallas_call(
        body,
        grid=(N // RB,),
        in_specs=[
            pl.BlockSpec((RB, 2), lambda i: (i, 0)),
            pl.BlockSpec((2, RB, 16), lambda i: (0, i, 0)),
            pl.BlockSpec((RB, 16), lambda i: (i, 0)),
            pl.BlockSpec((1, 16), lambda i: (0, 0)),
        ],
        out_specs=pl.BlockSpec((RB, 16), lambda i: (i, 0)),
        out_shape=jax.ShapeDtypeStruct((N, 16), jnp.float32),
    )(degT, q, h2p, b2)


def kernel(x, edge_index, W1, b1, W2, b2):
    pidx = jnp.arange(EP - E, dtype=jnp.int32)
    src2 = jnp.concatenate([edge_index[0], pidx % N]).reshape(EP // K, K)
    dst2 = jnp.concatenate([edge_index[1],
                            N + pidx % (NPAD - N)]).reshape(EP // K, K)
    degp = _sc_degree(dst2)
    degT = jnp.transpose(degp)[:N]
    h1p = _tc_layer1(degT, x, W1)
    p = _sc_agg64(h1p, src2, dst2)
    h2p = _tc_mid(degT, p, h1p, b1.reshape(1, 64), W2)
    q = _sc_agg16(h2p, src2, dst2)
    out = _tc_final(degT, q, h2p, b2.reshape(1, 16))
    return (out, edge_index)
